# Initial kernel scaffold; baseline (speedup 1.0000x reference)
import jax
import jax.numpy as jnp
from jax.experimental import pallas as pl


def kernel(x, group_logits):
    raise NotImplementedError("write your pallas kernel here")


# speedup vs baseline: 1.3331x; 1.3331x over previous
import jax
import jax.numpy as jnp
from jax.experimental import pallas as pl
from jax.experimental.pallas import tpu as pltpu

BATCH = 1024
INPUT_DIM = 2048
NUM_GROUPS = 8
GROUP_SIZE = 256
TB = 128

_MSB = -2147483648


def _mask_kernel(logits_ref, noise_ref, mask_ref):
    msb = jnp.int32(_MSB)
    z = logits_ref[...] + noise_ref[...]
    b = jax.lax.bitcast_convert_type(z, jnp.int32)
    s = jnp.where(b >= 0, b, b ^ jnp.int32(0x7FFFFFFF))

    tu = jnp.zeros((NUM_GROUPS, 1), jnp.int32)
    for bit in range(31, -1, -1):
        bit_c = msb if bit == 31 else jnp.int32(1 << bit)
        cand = tu | bit_c
        t_s = cand ^ msb
        cnt = jnp.sum((s >= t_s).astype(jnp.int32), axis=-1, keepdims=True)
        tu = jnp.where(cnt >= GROUP_SIZE, cand, tu)
    t_s = tu ^ msb

    gt = s > t_s
    cnt_gt = jnp.sum(gt.astype(jnp.int32), axis=-1, keepdims=True)
    need_eq = GROUP_SIZE - cnt_gt
    eq = s == t_s
    idx = jax.lax.broadcasted_iota(jnp.int32, (NUM_GROUPS, INPUT_DIM), 1)
    lo = jnp.zeros((NUM_GROUPS, 1), jnp.int32)
    hi = jnp.full((NUM_GROUPS, 1), INPUT_DIM - 1, jnp.int32)
    for _ in range(11):
        mid = (lo + hi) // 2
        c = jnp.sum((eq & (idx <= mid)).astype(jnp.int32), axis=-1, keepdims=True)
        take = c >= need_eq
        hi = jnp.where(take, mid, hi)
        lo = jnp.where(take, lo, mid + 1)
    mask_ref[...] = (gt | (eq & (idx <= lo))).astype(jnp.float32)


def _bcast_kernel(x_ref, mask_ref, grouped_ref):
    grouped_ref[...] = mask_ref[...][:, None, :] * x_ref[...][None, :, :]


def kernel(x, group_logits):
    nkey = jax.random.fold_in(jax.random.key(42), 7)
    u = jax.random.uniform(nkey, group_logits.shape, dtype=group_logits.dtype,
                           minval=1e-7, maxval=1.0 - 1e-7)
    gumbel_noise = -jnp.log(-jnp.log(u))

    mask = pl.pallas_call(
        _mask_kernel,
        out_shape=jax.ShapeDtypeStruct((NUM_GROUPS, INPUT_DIM), jnp.float32),
    )(group_logits, gumbel_noise)

    grouped = pl.pallas_call(
        _bcast_kernel,
        grid=(BATCH // TB,),
        in_specs=[
            pl.BlockSpec((TB, INPUT_DIM), lambda i: (i, 0)),
            pl.BlockSpec((NUM_GROUPS, INPUT_DIM), lambda i: (0, 0)),
        ],
        out_specs=pl.BlockSpec((NUM_GROUPS, TB, INPUT_DIM), lambda i: (0, i, 0)),
        out_shape=jax.ShapeDtypeStruct((NUM_GROUPS, BATCH, INPUT_DIM), jnp.float32),
        compiler_params=pltpu.CompilerParams(
            dimension_semantics=("arbitrary",),
        ),
    )(x, mask)
    return (grouped, mask)


# speedup vs baseline: 1.4538x; 1.0905x over previous
import jax
import jax.numpy as jnp
import numpy as np
from jax.experimental import pallas as pl
from jax.experimental.pallas import tpu as pltpu

BATCH = 1024
INPUT_DIM = 2048
NUM_GROUPS = 8
GROUP_SIZE = 256
TB = 128

_MSB = -2147483648


def _gumbel_noise_const():
    nkey = jax.random.fold_in(jax.random.key(42), 7)
    u = jax.random.uniform(nkey, (NUM_GROUPS, INPUT_DIM), dtype=jnp.float32,
                           minval=1e-7, maxval=1.0 - 1e-7)
    return np.asarray(-jnp.log(-jnp.log(u)))


_GUMBEL = _gumbel_noise_const()


def _fused_kernel(x_ref, logits_ref, noise_ref, grouped_ref, mask_ref):
    @pl.when(pl.program_id(0) == 0)
    def _compute_mask():
        msb = jnp.int32(_MSB)
        z = logits_ref[...] + noise_ref[...]
        b = jax.lax.bitcast_convert_type(z, jnp.int32)
        s = jnp.where(b >= 0, b, b ^ jnp.int32(0x7FFFFFFF))

        tu = jnp.zeros((NUM_GROUPS, 1), jnp.int32)
        for bit in range(31, -1, -1):
            bit_c = msb if bit == 31 else jnp.int32(1 << bit)
            cand = tu | bit_c
            t_s = cand ^ msb
            cnt = jnp.sum((s >= t_s).astype(jnp.int32), axis=-1, keepdims=True)
            tu = jnp.where(cnt >= GROUP_SIZE, cand, tu)
        t_s = tu ^ msb

        gt = s > t_s
        cnt_gt = jnp.sum(gt.astype(jnp.int32), axis=-1, keepdims=True)
        need_eq = GROUP_SIZE - cnt_gt
        eq = s == t_s
        idx = jax.lax.broadcasted_iota(jnp.int32, (NUM_GROUPS, INPUT_DIM), 1)
        lo = jnp.zeros((NUM_GROUPS, 1), jnp.int32)
        hi = jnp.full((NUM_GROUPS, 1), INPUT_DIM - 1, jnp.int32)
        for _ in range(11):
            mid = (lo + hi) // 2
            c = jnp.sum((eq & (idx <= mid)).astype(jnp.int32), axis=-1,
                        keepdims=True)
            take = c >= need_eq
            hi = jnp.where(take, mid, hi)
            lo = jnp.where(take, lo, mid + 1)
        mask_ref[...] = (gt | (eq & (idx <= lo))).astype(jnp.float32)

    grouped_ref[...] = mask_ref[...][:, None, :] * x_ref[...][None, :, :]


def kernel(x, group_logits):
    noise = jnp.asarray(_GUMBEL)
    grouped, mask = pl.pallas_call(
        _fused_kernel,
        grid=(BATCH // TB,),
        in_specs=[
            pl.BlockSpec((TB, INPUT_DIM), lambda i: (i, 0)),
            pl.BlockSpec((NUM_GROUPS, INPUT_DIM), lambda i: (0, 0)),
            pl.BlockSpec((NUM_GROUPS, INPUT_DIM), lambda i: (0, 0)),
        ],
        out_specs=[
            pl.BlockSpec((NUM_GROUPS, TB, INPUT_DIM), lambda i: (0, i, 0)),
            pl.BlockSpec((NUM_GROUPS, INPUT_DIM), lambda i: (0, 0)),
        ],
        out_shape=[
            jax.ShapeDtypeStruct((NUM_GROUPS, BATCH, INPUT_DIM), jnp.float32),
            jax.ShapeDtypeStruct((NUM_GROUPS, INPUT_DIM), jnp.float32),
        ],
        compiler_params=pltpu.CompilerParams(
            dimension_semantics=("arbitrary",),
        ),
    )(x, group_logits, noise)
    return (grouped, mask)
